# baseline (speedup 1.0000x reference)
import functools

import jax
import jax.numpy as jnp
from jax.experimental import pallas as pl
from jax.experimental.pallas import tpu as pltpu
from jax.experimental.pallas import tpu_sc as plsc

_K = 2048
_D = 64
_R = 2048
_NW = 32


def _argmin_kernel(zt_ref, wt_ref, idx_ref):
    ztb = zt_ref[0]
    wt = wt_ref[...]

    zb = ztb.T

    zsq = jnp.sum(zb * zb, axis=1, keepdims=True)
    wsq = jnp.sum(wt * wt, axis=0, keepdims=True)

    z2 = (zb.astype(jnp.bfloat16) * jnp.bfloat16(2.0))
    mm2 = jax.lax.dot_general(
        z2, wt, (((1,), (0,)), ((), ())),
        preferred_element_type=jnp.float32,
    )

    dist = (zsq - mm2) + wsq
    idx = jnp.argmin(dist, axis=1).astype(jnp.int32)
    idx_ref[0, 0, :] = idx


def _sc_gather_kernel(table_hbm, idx_hbm, out_hbm, idx_v, rows_v, sem):
    bpw, half = idx_v.shape[0], rows_v.shape[0]
    wid = jax.lax.axis_index("s") * 2 + jax.lax.axis_index("c")
    base = wid * bpw
    pltpu.sync_copy(idx_hbm.at[pl.ds(base, bpw)], idx_v)
    for h in range(bpw // half):
        pltpu.async_copy(
            table_hbm.at[idx_v.at[pl.ds(h * half, half)]], rows_v, sem
        ).wait()
        pltpu.sync_copy(rows_v, out_hbm.at[pl.ds(base + h * half, half)])


def _st_loss_kernel(zt_ref, xq_ref, xqt_ref, loss_ref):
    ztb = zt_ref[0]
    xqt = xq_ref[:, :_D].T
    xqt_ref[0] = ztb + (xqt - ztb)
    d = ztb - xqt
    loss_ref[...] = jnp.sum(d * d).reshape(1, 1, 1)


def _tc_argmin(ztc, wt, tiles):
    return pl.pallas_call(
        _argmin_kernel,
        grid=(tiles,),
        in_specs=[
            pl.BlockSpec((1, _D, _R), lambda i: (0, 0, i)),
            pl.BlockSpec((_D, _K), lambda i: (0, 0)),
        ],
        out_specs=pl.BlockSpec((1, 1, _R), lambda i: (i, 0, 0)),
        out_shape=jax.ShapeDtypeStruct((tiles, 1, _R), jnp.int32),
        compiler_params=pltpu.CompilerParams(
            dimension_semantics=("arbitrary",),
        ),
    )(ztc, wt)


def _sc_gather(wpad, idx_flat, n):
    bpw = n // _NW
    mesh = plsc.VectorSubcoreMesh(core_axis_name="c", subcore_axis_name="s")
    return functools.partial(
        pl.kernel,
        mesh=mesh,
        out_type=jax.ShapeDtypeStruct((n, 128), jnp.float32),
        scratch_types=[
            pltpu.VMEM((bpw,), jnp.int32),
            pltpu.VMEM((bpw // 2, 128), jnp.float32),
            pltpu.SemaphoreType.DMA,
        ],
    )(_sc_gather_kernel)(wpad, idx_flat)


def _tc_st_loss(ztc, xq, tiles):
    return pl.pallas_call(
        _st_loss_kernel,
        grid=(tiles,),
        in_specs=[
            pl.BlockSpec((1, _D, _R), lambda i: (0, 0, i)),
            pl.BlockSpec((_R, 128), lambda i: (i, 0)),
        ],
        out_specs=[
            pl.BlockSpec((1, _D, _R), lambda i: (0, 0, i)),
            pl.BlockSpec((1, 1, 1), lambda i: (i, 0, 0)),
        ],
        out_shape=[
            jax.ShapeDtypeStruct((1, _D, ztc.shape[2]), jnp.float32),
            jax.ShapeDtypeStruct((tiles, 1, 1), jnp.float32),
        ],
        compiler_params=pltpu.CompilerParams(
            dimension_semantics=("arbitrary",),
        ),
    )(ztc, xq)


def kernel(z, codebook):
    B, T, D = z.shape
    zt = jnp.transpose(z, (0, 2, 1))
    tiles = T // _R
    wt = codebook.T
    wpad = jnp.pad(codebook, ((0, 0), (0, 128 - D)))

    idx_chunks = []
    xq_chunks = []
    for b in range(B):
        ztc = jax.lax.slice_in_dim(zt, b, b + 1, axis=0)
        idx3 = _tc_argmin(ztc, wt, tiles)
        idx_chunks.append(idx3)
        xq_chunks.append(_sc_gather(wpad, idx3.reshape(T), T))

    xqt_chunks = []
    loss_chunks = []
    for b in range(B):
        ztc = jax.lax.slice_in_dim(zt, b, b + 1, axis=0)
        xqt_b, loss_b = _tc_st_loss(ztc, xq_chunks[b], tiles)
        xqt_chunks.append(xqt_b)
        loss_chunks.append(loss_b)

    xqt = jnp.concatenate(xqt_chunks, axis=0)
    xq_st = jnp.transpose(xqt, (0, 2, 1))
    idx = jnp.concatenate(idx_chunks, axis=0).reshape(B, T)
    commit_loss = (
        jnp.sum(jnp.concatenate(loss_chunks)) * jnp.float32(2.0 ** -21)
    )
    return xq_st, idx, commit_loss
